# Initial kernel scaffold; baseline (speedup 1.0000x reference)
import functools

import jax
import jax.numpy as jnp
from jax import lax
from jax.experimental import pallas as pl
from jax.experimental.pallas import tpu as pltpu
from jax.experimental.pallas import tpu_sc as plsc

N = 10000
E = 160000
FC = 128
NC = 2
NS = 16
EPT = E // NS
GK = 125
G = EPT // GK
RPT = N // NS
ZR = 125


def _zero_rows(ref, nrows, ncols):
    zv = jnp.zeros((16,), jnp.float32)

    def body(i, _):
        for j in range(ncols // 16):
            ref[i, pl.ds(j * 16, 16)] = zv
        return 0

    lax.fori_loop(0, nrows, body, 0)


def _fill_ones(ref, nrows, ncols):
    ov = jnp.ones((16,), jnp.float32)

    def body(i, _):
        for j in range(ncols // 16):
            ref[i, pl.ds(j * 16, 16)] = ov
        return 0

    lax.fori_loop(0, nrows, body, 0)


def _make_hist():
    mesh = plsc.VectorSubcoreMesh(core_axis_name="c", subcore_axis_name="s")

    @functools.partial(
        pl.kernel,
        mesh=mesh,
        out_type=jax.ShapeDtypeStruct((2, N, 16), jnp.float32),
        scratch_types=[
            pltpu.VMEM((G, GK), jnp.int32),
            pltpu.VMEM((ZR, 16), jnp.float32),
            pltpu.VMEM((ZR, 16), jnp.float32),
            pltpu.VMEM_SHARED((N, 16), jnp.float32),
        ],
    )
    def hist(ei_hbm, deg_hbm, idx_v, ones_v, zero_v, hist_s):
        cid = lax.axis_index("c")
        sid = lax.axis_index("s")
        pltpu.sync_copy(ei_hbm.at[cid, sid], idx_v)
        _fill_ones(ones_v, ZR, 16)
        _zero_rows(zero_v, ZR, 16)
        for t in range(RPT // ZR):
            pltpu.sync_copy(zero_v, hist_s.at[pl.ds(sid * RPT + t * ZR, ZR)])
        plsc.subcore_barrier()

        def body(g, _):
            pltpu.sync_copy(ones_v, hist_s.at[idx_v.at[g]], add=True)
            return 0

        lax.fori_loop(0, G, body, 0)
        plsc.subcore_barrier()
        pltpu.sync_copy(
            hist_s.at[pl.ds(sid * RPT, RPT)],
            deg_hbm.at[cid, pl.ds(sid * RPT, RPT)],
        )

    return hist


def _make_spmm(C):
    CPC = C // NC
    mesh = plsc.VectorSubcoreMesh(core_axis_name="c", subcore_axis_name="s")

    @functools.partial(
        pl.kernel,
        mesh=mesh,
        out_type=jax.ShapeDtypeStruct((C, N, FC), jnp.float32),
        scratch_types=[
            pltpu.VMEM((G, GK), jnp.int32),
            pltpu.VMEM((G, GK), jnp.int32),
            pltpu.VMEM((2, GK, FC), jnp.float32),
            pltpu.VMEM((ZR, FC), jnp.float32),
            pltpu.VMEM_SHARED((N, FC), jnp.float32),
            pltpu.SemaphoreType.DMA,
            pltpu.SemaphoreType.DMA,
        ],
    )
    def spmm(h_hbm, ei_hbm, out_hbm, sidx, didx, buf, zero_v, agg, sem0, sem1):
        cid = lax.axis_index("c")
        sid = lax.axis_index("s")
        pltpu.sync_copy(ei_hbm.at[0, sid], sidx)
        pltpu.sync_copy(ei_hbm.at[1, sid], didx)
        _zero_rows(zero_v, ZR, FC)
        sems = [sem0, sem1]

        for lc in range(CPC):
            c = cid * CPC + lc
            for t in range(RPT // ZR):
                pltpu.sync_copy(zero_v, agg.at[pl.ds(sid * RPT + t * ZR, ZR)])
            plsc.subcore_barrier()

            pltpu.async_copy(h_hbm.at[c].at[sidx.at[0]], buf.at[0], sems[0])

            def body(g, _):
                b = lax.rem(g, 2)
                bn = lax.rem(g + 1, 2)
                pltpu.async_copy(
                    h_hbm.at[c].at[sidx.at[g + 1]], buf.at[bn], sems[1]
                )
                pltpu.make_async_copy(
                    h_hbm.at[c].at[sidx.at[g]], buf.at[b], sems[0]
                ).wait()
                pltpu.sync_copy(buf.at[b], agg.at[didx.at[g]], add=True)
                return 0

            def body2(g, _):
                b = lax.rem(g, 2)
                bn = lax.rem(g + 1, 2)
                pltpu.async_copy(
                    h_hbm.at[c].at[sidx.at[g + 1]], buf.at[bn], sems[1]
                )
                pltpu.make_async_copy(
                    h_hbm.at[c].at[sidx.at[g]], buf.at[b], sems[0]
                ).wait()
                pltpu.sync_copy(buf.at[b], agg.at[didx.at[g]], add=True)
                return 0

            lax.fori_loop(0, G - 1, body2, 0)
            pltpu.make_async_copy(
                h_hbm.at[c].at[sidx.at[G - 1]], buf.at[(G - 1) % 2], sems[0]
            ).wait()
            pltpu.sync_copy(buf.at[(G - 1) % 2], agg.at[didx.at[G - 1]], add=True)

            plsc.subcore_barrier()
            pltpu.sync_copy(
                agg.at[pl.ds(sid * RPT, RPT)],
                out_hbm.at[c, pl.ds(sid * RPT, RPT)],
            )
            if lc + 1 < CPC:
                plsc.subcore_barrier()

    return spmm


RB = 1000


def _prep_body(deg_ref, x_ref, ns_ref, nd_ref, p0_ref):
    ns = lax.rsqrt(jnp.maximum(deg_ref[0, :, 0:1], 1.0))
    nd = lax.rsqrt(jnp.maximum(deg_ref[1, :, 0:1], 1.0))
    ns_ref[...] = ns
    nd_ref[...] = nd
    x = x_ref[...] * ns
    p0_ref[0] = x[:, :FC]
    p0_ref[1] = x[:, FC:]


def _prep(deg, features):
    grid = (N // RB,)
    return pl.pallas_call(
        _prep_body,
        grid=grid,
        in_specs=[
            pl.BlockSpec((2, RB, 16), lambda i: (0, i, 0)),
            pl.BlockSpec((RB, 256), lambda i: (i, 0)),
        ],
        out_specs=[
            pl.BlockSpec((RB, 1), lambda i: (i, 0)),
            pl.BlockSpec((RB, 1), lambda i: (i, 0)),
            pl.BlockSpec((2, RB, FC), lambda i: (0, i, 0)),
        ],
        out_shape=[
            jax.ShapeDtypeStruct((N, 1), jnp.float32),
            jax.ShapeDtypeStruct((N, 1), jnp.float32),
            jax.ShapeDtypeStruct((2, N, FC), jnp.float32),
        ],
    )(deg, features)


def _mm_body(nd_ref, x_ref, w_ref, b_ref, ns_ref, o_ref, *, nk, relu, scaled, bias):
    k = pl.program_id(2)
    x = x_ref[0]
    if scaled:
        x = x * nd_ref[...]
    acc = jnp.dot(x, w_ref[0], preferred_element_type=jnp.float32)

    @pl.when(k == 0)
    def _():
        o_ref[0] = acc

    @pl.when(k > 0)
    def _():
        o_ref[0] += acc

    @pl.when(k == nk - 1)
    def _():
        r = o_ref[0]
        if bias:
            r = r + b_ref[...]
        if relu:
            r = jnp.maximum(r, 0.0)
        if scaled:
            r = r * ns_ref[...]
        o_ref[0] = r


def _mm(x, w, b, nd, ns, relu, scaled, bias):
    cin = x.shape[0]
    h = w.shape[2]
    cout = h // FC
    grid = (N // RB, cout, cin)
    body = functools.partial(_mm_body, nk=cin, relu=relu, scaled=scaled, bias=bias)
    return pl.pallas_call(
        body,
        grid=grid,
        in_specs=[
            pl.BlockSpec((RB, 1), lambda i, j, k: (i, 0)),
            pl.BlockSpec((1, RB, FC), lambda i, j, k: (k, i, 0)),
            pl.BlockSpec((1, FC, FC), lambda i, j, k: (k, 0, j)),
            pl.BlockSpec((1, FC), lambda i, j, k: (0, j)),
            pl.BlockSpec((RB, 1), lambda i, j, k: (i, 0)),
        ],
        out_specs=pl.BlockSpec((1, RB, FC), lambda i, j, k: (j, i, 0)),
        out_shape=jax.ShapeDtypeStruct((cout, N, FC), jnp.float32),
    )(nd, x, w.reshape(cin, FC, h), b.reshape(1, h), ns)


def _final_body(a_ref, nd_ref, b_ref, o_ref):
    o_ref[...] = jnp.maximum(a_ref[0] * nd_ref[...] + b_ref[...], 0.0)


def _final(a3, nd, b3):
    grid = (N // RB, 2)
    return pl.pallas_call(
        _final_body,
        grid=grid,
        in_specs=[
            pl.BlockSpec((1, RB, FC), lambda i, j: (j, i, 0)),
            pl.BlockSpec((RB, 1), lambda i, j: (i, 0)),
            pl.BlockSpec((1, FC), lambda i, j: (0, j)),
        ],
        out_specs=pl.BlockSpec((RB, FC), lambda i, j: (i, j)),
        out_shape=jax.ShapeDtypeStruct((N, 256), jnp.float32),
    )(a3, nd, b3.reshape(1, 256))


_hist_k = _make_hist()
_spmm2 = _make_spmm(2)
_spmm4 = _make_spmm(4)


@jax.jit
def kernel(features, edge_index, W1, b1, W2, b2, W3, b3):
    ei = edge_index.reshape(2, NS, G, GK)
    deg = _hist_k(ei)
    ns, nd, p0 = _prep(deg, features)

    a1 = _spmm2(p0, ei)
    h1 = _mm(a1, W1.reshape(2, FC, 512), b1, nd, ns, relu=True, scaled=True, bias=True)
    a2 = _spmm4(h1, ei)
    q2 = _mm(a2, W2.reshape(4, FC, 512), b2, nd, ns, relu=True, scaled=True, bias=True)
    p2 = _mm(q2, W3.reshape(4, FC, 256), b3, nd, ns, relu=False, scaled=False, bias=False)
    a3 = _spmm2(p2, ei)
    return _final(a3, nd, b3)


# speedup vs baseline: 6.2387x; 6.2387x over previous
import functools

import jax
import jax.numpy as jnp
from jax import lax
from jax.experimental import pallas as pl
from jax.experimental.pallas import tpu as pltpu
from jax.experimental.pallas import tpu_sc as plsc

N = 10000
E = 160000
FC = 128
NC = 2
NS = 16
EPT = E // NS
GK = 100
G = EPT // GK
SEG = 50
NSEG = G // SEG
ZSP = 25
WPT = 624


def _zero_rows(ref, nrows, ncols):
    zv = jnp.zeros((16,), jnp.float32)

    def body(i, _):
        for j in range(ncols // 16):
            ref[i, pl.ds(j * 16, 16)] = zv
        return 0

    lax.fori_loop(0, nrows, body, 0)


def _fill_ones(ref, nrows, ncols):
    ov = jnp.ones((16,), jnp.float32)

    def body(i, _):
        for j in range(ncols // 16):
            ref[i, pl.ds(j * 16, 16)] = ov
        return 0

    lax.fori_loop(0, nrows, body, 0)


def _zero_shared(zero_v, shared, sid):
    def zbody(t, _):
        pltpu.sync_copy(zero_v, shared.at[pl.ds(sid * 625 + t * ZSP, ZSP)])
        return 0

    lax.fori_loop(0, 625 // ZSP, zbody, 0)


def _writeout_shared(shared, out2d, sid):
    pltpu.sync_copy(
        shared.at[pl.ds(sid * WPT, WPT)], out2d.at[pl.ds(sid * WPT, WPT)]
    )

    @pl.when(sid == NS - 1)
    def _():
        pltpu.sync_copy(
            shared.at[pl.ds(NS * WPT, N - NS * WPT)],
            out2d.at[pl.ds(NS * WPT, N - NS * WPT)],
        )


def _make_hist():
    mesh = plsc.VectorSubcoreMesh(core_axis_name="c", subcore_axis_name="s")

    @functools.partial(
        pl.kernel,
        mesh=mesh,
        out_type=jax.ShapeDtypeStruct((2, N, FC), jnp.float32),
        scratch_types=[
            pltpu.VMEM((NSEG, SEG, GK), jnp.int32),
            pltpu.VMEM((GK, FC), jnp.float32),
            pltpu.VMEM((ZSP, FC), jnp.float32),
            pltpu.VMEM_SHARED((N, FC), jnp.float32),
        ],
    )
    def hist(ei_hbm, deg_hbm, idx_v, ones_v, zero_v, hist_s):
        cid = lax.axis_index("c")
        sid = lax.axis_index("s")
        pltpu.sync_copy(ei_hbm.at[cid, sid], idx_v)
        _fill_ones(ones_v, GK, FC)
        _zero_rows(zero_v, ZSP, FC)
        _zero_shared(zero_v, hist_s, sid)
        plsc.subcore_barrier()

        def body(g, _):
            pltpu.sync_copy(
                ones_v, hist_s.at[idx_v.at[g // SEG, g % SEG]], add=True
            )
            return 0

        lax.fori_loop(0, G, body, 0)
        plsc.subcore_barrier()
        _writeout_shared(hist_s, deg_hbm.at[cid], sid)

    return hist


def _make_spmm(C):
    CPC = C // NC
    mesh = plsc.VectorSubcoreMesh(core_axis_name="c", subcore_axis_name="s")

    @functools.partial(
        pl.kernel,
        mesh=mesh,
        out_type=jax.ShapeDtypeStruct((C, N, FC), jnp.float32),
        scratch_types=[
            pltpu.VMEM((SEG, GK), jnp.int32),
            pltpu.VMEM((SEG, GK), jnp.int32),
            pltpu.VMEM((2, GK, FC), jnp.float32),
            pltpu.VMEM((ZSP, FC), jnp.float32),
            pltpu.VMEM_SHARED((N, FC), jnp.float32),
            pltpu.SemaphoreType.DMA,
            pltpu.SemaphoreType.DMA,
        ],
    )
    def spmm(h_hbm, ei_hbm, out_hbm, sidx, didx, buf, zero_v, agg, sem0, sem1):
        cid = lax.axis_index("c")
        sid = lax.axis_index("s")
        _zero_rows(zero_v, ZSP, FC)

        for lc in range(CPC):
            c = cid * CPC + lc
            _zero_shared(zero_v, agg, sid)
            plsc.subcore_barrier()

            def gat(g, slot, sem):
                return pltpu.async_copy(
                    h_hbm.at[c].at[sidx.at[g]], buf.at[slot], sem
                )

            def wait(g, slot, sem):
                pltpu.make_async_copy(
                    h_hbm.at[c].at[sidx.at[g]], buf.at[slot], sem
                ).wait()

            def sca(g, slot):
                pltpu.sync_copy(buf.at[slot], agg.at[didx.at[g]], add=True)

            def segbody(seg, _):
                pltpu.sync_copy(ei_hbm.at[0, sid, seg], sidx)
                pltpu.sync_copy(ei_hbm.at[1, sid, seg], didx)
                gat(0, 0, sem0)

                def body(gg, _):
                    g0 = 2 * gg
                    gat(g0 + 1, 1, sem1)
                    wait(g0, 0, sem0)
                    sca(g0, 0)
                    gat(g0 + 2, 0, sem0)
                    wait(g0 + 1, 1, sem1)
                    sca(g0 + 1, 1)
                    return 0

                lax.fori_loop(0, SEG // 2 - 1, body, 0)
                gat(SEG - 1, 1, sem1)
                wait(SEG - 2, 0, sem0)
                sca(SEG - 2, 0)
                wait(SEG - 1, 1, sem1)
                sca(SEG - 1, 1)
                return 0

            lax.fori_loop(0, NSEG, segbody, 0)

            plsc.subcore_barrier()
            _writeout_shared(agg, out_hbm.at[c], sid)
            if lc + 1 < CPC:
                plsc.subcore_barrier()

    return spmm


RB = 1000


def _prep_body(deg_ref, x_ref, ns_ref, nd_ref, p0_ref):
    ns = lax.rsqrt(jnp.maximum(deg_ref[0, :, 0:1], 1.0))
    nd = lax.rsqrt(jnp.maximum(deg_ref[1, :, 0:1], 1.0))
    ns_ref[...] = ns
    nd_ref[...] = nd
    x = x_ref[...] * ns
    p0_ref[0] = x[:, :FC]
    p0_ref[1] = x[:, FC:]


def _prep(deg, features):
    grid = (N // RB,)
    return pl.pallas_call(
        _prep_body,
        grid=grid,
        in_specs=[
            pl.BlockSpec((2, RB, FC), lambda i: (0, i, 0)),
            pl.BlockSpec((RB, 256), lambda i: (i, 0)),
        ],
        out_specs=[
            pl.BlockSpec((RB, 1), lambda i: (i, 0)),
            pl.BlockSpec((RB, 1), lambda i: (i, 0)),
            pl.BlockSpec((2, RB, FC), lambda i: (0, i, 0)),
        ],
        out_shape=[
            jax.ShapeDtypeStruct((N, 1), jnp.float32),
            jax.ShapeDtypeStruct((N, 1), jnp.float32),
            jax.ShapeDtypeStruct((2, N, FC), jnp.float32),
        ],
    )(deg, features)


def _mm_body(nd_ref, x_ref, w_ref, b_ref, ns_ref, o_ref, *, nk, relu, scaled, bias):
    k = pl.program_id(2)
    x = x_ref[0]
    if scaled:
        x = x * nd_ref[...]
    acc = jnp.dot(x, w_ref[0], preferred_element_type=jnp.float32)

    @pl.when(k == 0)
    def _():
        o_ref[0] = acc

    @pl.when(k > 0)
    def _():
        o_ref[0] += acc

    @pl.when(k == nk - 1)
    def _():
        r = o_ref[0]
        if bias:
            r = r + b_ref[...]
        if relu:
            r = jnp.maximum(r, 0.0)
        if scaled:
            r = r * ns_ref[...]
        o_ref[0] = r


def _mm(x, w, b, nd, ns, relu, scaled, bias):
    cin = x.shape[0]
    h = w.shape[2]
    cout = h // FC
    grid = (N // RB, cout, cin)
    body = functools.partial(_mm_body, nk=cin, relu=relu, scaled=scaled, bias=bias)
    return pl.pallas_call(
        body,
        grid=grid,
        in_specs=[
            pl.BlockSpec((RB, 1), lambda i, j, k: (i, 0)),
            pl.BlockSpec((1, RB, FC), lambda i, j, k: (k, i, 0)),
            pl.BlockSpec((1, FC, FC), lambda i, j, k: (k, 0, j)),
            pl.BlockSpec((1, FC), lambda i, j, k: (0, j)),
            pl.BlockSpec((RB, 1), lambda i, j, k: (i, 0)),
        ],
        out_specs=pl.BlockSpec((1, RB, FC), lambda i, j, k: (j, i, 0)),
        out_shape=jax.ShapeDtypeStruct((cout, N, FC), jnp.float32),
    )(nd, x, w, b.reshape(1, h), ns)


def _final_body(a_ref, nd_ref, b_ref, o_ref):
    o_ref[...] = jnp.maximum(a_ref[0] * nd_ref[...] + b_ref[...], 0.0)


def _final(a3, nd, b3):
    grid = (N // RB, 2)
    return pl.pallas_call(
        _final_body,
        grid=grid,
        in_specs=[
            pl.BlockSpec((1, RB, FC), lambda i, j: (j, i, 0)),
            pl.BlockSpec((RB, 1), lambda i, j: (i, 0)),
            pl.BlockSpec((1, FC), lambda i, j: (0, j)),
        ],
        out_specs=pl.BlockSpec((RB, FC), lambda i, j: (i, j)),
        out_shape=jax.ShapeDtypeStruct((N, 256), jnp.float32),
    )(a3, nd, b3.reshape(1, 256))


_hist_k = _make_hist()
_spmm2 = _make_spmm(2)
_spmm4 = _make_spmm(4)


@jax.jit
def kernel(features, edge_index, W1, b1, W2, b2, W3, b3):
    ei = edge_index.reshape(2, NS, NSEG, SEG, GK)
    deg = _hist_k(ei)
    ns, nd, p0 = _prep(deg, features)

    a1 = _spmm2(p0, ei)
    h1 = _mm(a1, W1.reshape(2, FC, 512), b1, nd, ns, relu=True, scaled=True, bias=True)
    a2 = _spmm4(h1, ei)
    q2 = _mm(a2, W2.reshape(4, FC, 512), b2, nd, ns, relu=True, scaled=True, bias=True)
    p2 = _mm(q2, W3.reshape(4, FC, 256), b3, nd, ns, relu=False, scaled=False, bias=False)
    a3 = _spmm2(p2, ei)
    return _final(a3, nd, b3)


# speedup vs baseline: 8.7552x; 1.4034x over previous
import functools

import jax
import jax.numpy as jnp
from jax import lax
from jax.experimental import pallas as pl
from jax.experimental.pallas import tpu as pltpu
from jax.experimental.pallas import tpu_sc as plsc

N = 10000
E = 160000
FC = 128
NC = 2
NS = 16
EPT = E // NS
GK = 100
G = EPT // GK
SEG = 50
NSEG = G // SEG
ZSP = 25
WPT = 624


def _zero_rows(ref, nrows, ncols):
    zv = jnp.zeros((16,), jnp.float32)

    def body(i, _):
        for j in range(ncols // 16):
            ref[i, pl.ds(j * 16, 16)] = zv
        return 0

    lax.fori_loop(0, nrows, body, 0)


def _fill_ones(ref, nrows, ncols):
    ov = jnp.ones((16,), jnp.float32)

    def body(i, _):
        for j in range(ncols // 16):
            ref[i, pl.ds(j * 16, 16)] = ov
        return 0

    lax.fori_loop(0, nrows, body, 0)


def _zero_shared(zero_v, shared, sid):
    def zbody(t, _):
        pltpu.sync_copy(zero_v, shared.at[pl.ds(sid * 625 + t * ZSP, ZSP)])
        return 0

    lax.fori_loop(0, 625 // ZSP, zbody, 0)


def _writeout_shared(shared, out2d, sid):
    pltpu.sync_copy(
        shared.at[pl.ds(sid * WPT, WPT)], out2d.at[pl.ds(sid * WPT, WPT)]
    )

    @pl.when(sid == NS - 1)
    def _():
        pltpu.sync_copy(
            shared.at[pl.ds(NS * WPT, N - NS * WPT)],
            out2d.at[pl.ds(NS * WPT, N - NS * WPT)],
        )


def _make_hist():
    mesh = plsc.VectorSubcoreMesh(core_axis_name="c", subcore_axis_name="s")

    @functools.partial(
        pl.kernel,
        mesh=mesh,
        out_type=jax.ShapeDtypeStruct((2, N, FC), jnp.float32),
        scratch_types=[
            pltpu.VMEM((NSEG, SEG, GK), jnp.int32),
            pltpu.VMEM((GK, FC), jnp.float32),
            pltpu.VMEM((ZSP, FC), jnp.float32),
            pltpu.VMEM_SHARED((N, FC), jnp.float32),
        ],
    )
    def hist(ei_hbm, deg_hbm, idx_v, ones_v, zero_v, hist_s):
        cid = lax.axis_index("c")
        sid = lax.axis_index("s")
        pltpu.sync_copy(ei_hbm.at[cid, sid], idx_v)
        _fill_ones(ones_v, GK, FC)
        _zero_rows(zero_v, ZSP, FC)
        _zero_shared(zero_v, hist_s, sid)
        plsc.subcore_barrier()

        def body(g, _):
            pltpu.sync_copy(
                ones_v, hist_s.at[idx_v.at[g // SEG, g % SEG]], add=True
            )
            return 0

        lax.fori_loop(0, G, body, 0)
        plsc.subcore_barrier()
        _writeout_shared(hist_s, deg_hbm.at[cid], sid)

    return hist


def _make_spmm(C):
    CPC = C // NC
    mesh = plsc.VectorSubcoreMesh(core_axis_name="c", subcore_axis_name="s")

    @functools.partial(
        pl.kernel,
        mesh=mesh,
        out_type=jax.ShapeDtypeStruct((C, N, FC), jnp.float32),
        scratch_types=[
            pltpu.VMEM((SEG, GK), jnp.int32),
            pltpu.VMEM((SEG, GK), jnp.int32),
            pltpu.VMEM((2, GK, FC), jnp.float32),
            pltpu.VMEM((ZSP, FC), jnp.float32),
            pltpu.VMEM_SHARED((N, FC), jnp.float32),
            pltpu.SemaphoreType.DMA,
            pltpu.SemaphoreType.DMA,
        ],
    )
    def spmm(h_hbm, ei_hbm, out_hbm, sidx, didx, buf, zero_v, agg, sem0, sem1):
        cid = lax.axis_index("c")
        sid = lax.axis_index("s")
        _zero_rows(zero_v, ZSP, FC)

        for lc in range(CPC):
            c = cid * CPC + lc
            _zero_shared(zero_v, agg, sid)
            plsc.subcore_barrier()

            def gat(g, slot, sem):
                return pltpu.async_copy(
                    h_hbm.at[c].at[sidx.at[g]], buf.at[slot], sem
                )

            def wait(g, slot, sem):
                pltpu.make_async_copy(
                    h_hbm.at[c].at[sidx.at[g]], buf.at[slot], sem
                ).wait()

            def sca(g, slot):
                pltpu.sync_copy(buf.at[slot], agg.at[didx.at[g]], add=True)

            def segbody(seg, _):
                pltpu.sync_copy(ei_hbm.at[0, sid, seg], sidx)
                pltpu.sync_copy(ei_hbm.at[1, sid, seg], didx)
                gat(0, 0, sem0)

                def body(gg, _):
                    g0 = 2 * gg
                    gat(g0 + 1, 1, sem1)
                    wait(g0, 0, sem0)
                    sca(g0, 0)
                    gat(g0 + 2, 0, sem0)
                    wait(g0 + 1, 1, sem1)
                    sca(g0 + 1, 1)
                    return 0

                lax.fori_loop(0, SEG // 2 - 1, body, 0)
                gat(SEG - 1, 1, sem1)
                wait(SEG - 2, 0, sem0)
                sca(SEG - 2, 0)
                wait(SEG - 1, 1, sem1)
                sca(SEG - 1, 1)
                return 0

            lax.fori_loop(0, NSEG, segbody, 0)

            plsc.subcore_barrier()
            _writeout_shared(agg, out_hbm.at[c], sid)
            if lc + 1 < CPC:
                plsc.subcore_barrier()

    return spmm


RB = 1000


def _prep_body(deg_ref, x_ref, ns_ref, nd_ref, p0_ref):
    ns = lax.rsqrt(jnp.maximum(deg_ref[0, :, 0:1], 1.0))
    nd = lax.rsqrt(jnp.maximum(deg_ref[1, :, 0:1], 1.0))
    ns_ref[...] = ns
    nd_ref[...] = nd
    x = x_ref[...] * ns
    p0_ref[0] = x[:, :FC]
    p0_ref[1] = x[:, FC:]


def _prep(deg, features):
    grid = (N // RB,)
    return pl.pallas_call(
        _prep_body,
        grid=grid,
        in_specs=[
            pl.BlockSpec((2, RB, FC), lambda i: (0, i, 0)),
            pl.BlockSpec((RB, 256), lambda i: (i, 0)),
        ],
        out_specs=[
            pl.BlockSpec((RB, 1), lambda i: (i, 0)),
            pl.BlockSpec((RB, 1), lambda i: (i, 0)),
            pl.BlockSpec((2, RB, FC), lambda i: (0, i, 0)),
        ],
        out_shape=[
            jax.ShapeDtypeStruct((N, 1), jnp.float32),
            jax.ShapeDtypeStruct((N, 1), jnp.float32),
            jax.ShapeDtypeStruct((2, N, FC), jnp.float32),
        ],
    )(deg, features)


def _mm_body(nd_ref, x_ref, w_ref, b_ref, ns_ref, o_ref, *, cin, cout, relu,
             scaled, bias):
    x = jnp.concatenate([x_ref[i] for i in range(cin)], axis=1)
    if scaled:
        x = x * nd_ref[...]
    y = jnp.dot(x, w_ref[...], preferred_element_type=jnp.float32)
    if bias:
        y = y + b_ref[...]
    if relu:
        y = jnp.maximum(y, 0.0)
    if scaled:
        y = y * ns_ref[...]
    for j in range(cout):
        o_ref[j] = y[:, j * FC:(j + 1) * FC]


def _mm(x, w, b, nd, ns, relu, scaled, bias):
    cin = x.shape[0]
    k, h = w.shape
    cout = h // FC
    grid = (N // RB,)
    body = functools.partial(_mm_body, cin=cin, cout=cout, relu=relu,
                             scaled=scaled, bias=bias)
    return pl.pallas_call(
        body,
        grid=grid,
        in_specs=[
            pl.BlockSpec((RB, 1), lambda i: (i, 0)),
            pl.BlockSpec((cin, RB, FC), lambda i: (0, i, 0)),
            pl.BlockSpec((k, h), lambda i: (0, 0)),
            pl.BlockSpec((1, h), lambda i: (0, 0)),
            pl.BlockSpec((RB, 1), lambda i: (i, 0)),
        ],
        out_specs=pl.BlockSpec((cout, RB, FC), lambda i: (0, i, 0)),
        out_shape=jax.ShapeDtypeStruct((cout, N, FC), jnp.float32),
    )(nd, x, w, b.reshape(1, h), ns)


def _mm23_body(nd_ref, x_ref, w2_ref, b2_ref, ns_ref, w3_ref, o_ref, *, cin,
               cout):
    x = jnp.concatenate([x_ref[i] for i in range(cin)], axis=1)
    x = x * nd_ref[...]
    z = jnp.dot(x, w2_ref[...], preferred_element_type=jnp.float32)
    z = jnp.maximum(z + b2_ref[...], 0.0) * ns_ref[...]
    y = jnp.dot(z, w3_ref[...], preferred_element_type=jnp.float32)
    for j in range(cout):
        o_ref[j] = y[:, j * FC:(j + 1) * FC]


def _mm23(x, w2, b2, w3, nd, ns):
    cin = x.shape[0]
    k, h = w2.shape
    h3 = w3.shape[1]
    cout = h3 // FC
    grid = (N // RB,)
    body = functools.partial(_mm23_body, cin=cin, cout=cout)
    return pl.pallas_call(
        body,
        grid=grid,
        in_specs=[
            pl.BlockSpec((RB, 1), lambda i: (i, 0)),
            pl.BlockSpec((cin, RB, FC), lambda i: (0, i, 0)),
            pl.BlockSpec((k, h), lambda i: (0, 0)),
            pl.BlockSpec((1, h), lambda i: (0, 0)),
            pl.BlockSpec((RB, 1), lambda i: (i, 0)),
            pl.BlockSpec((h, h3), lambda i: (0, 0)),
        ],
        out_specs=pl.BlockSpec((cout, RB, FC), lambda i: (0, i, 0)),
        out_shape=jax.ShapeDtypeStruct((cout, N, FC), jnp.float32),
    )(nd, x, w2, b2.reshape(1, h), ns, w3)


def _final_body(a_ref, nd_ref, b_ref, o_ref):
    o_ref[...] = jnp.maximum(a_ref[0] * nd_ref[...] + b_ref[...], 0.0)


def _final(a3, nd, b3):
    grid = (N // RB, 2)
    return pl.pallas_call(
        _final_body,
        grid=grid,
        in_specs=[
            pl.BlockSpec((1, RB, FC), lambda i, j: (j, i, 0)),
            pl.BlockSpec((RB, 1), lambda i, j: (i, 0)),
            pl.BlockSpec((1, FC), lambda i, j: (0, j)),
        ],
        out_specs=pl.BlockSpec((RB, FC), lambda i, j: (i, j)),
        out_shape=jax.ShapeDtypeStruct((N, 256), jnp.float32),
    )(a3, nd, b3.reshape(1, 256))


_hist_k = _make_hist()
_spmm2 = _make_spmm(2)
_spmm4 = _make_spmm(4)


@jax.jit
def kernel(features, edge_index, W1, b1, W2, b2, W3, b3):
    ei = edge_index.reshape(2, NS, NSEG, SEG, GK)
    deg = _hist_k(ei)
    ns, nd, p0 = _prep(deg, features)

    a1 = _spmm2(p0, ei)
    h1 = _mm(a1, W1, b1, nd, ns, relu=True, scaled=True, bias=True)
    a2 = _spmm4(h1, ei)
    p2 = _mm23(a2, W2, b2, W3, nd, ns)
    a3 = _spmm2(p2, ei)
    return _final(a3, nd, b3)


# speedup vs baseline: 9.3327x; 1.0660x over previous
import functools

import jax
import jax.numpy as jnp
from jax import lax
from jax.experimental import pallas as pl
from jax.experimental.pallas import tpu as pltpu
from jax.experimental.pallas import tpu_sc as plsc

N = 10000
E = 160000
FC = 128
NC = 2
NS = 16
EPT = E // NS
GK = 100
G = EPT // GK
SEG = 50
NSEG = G // SEG
ZSP = 25
WPT = 624


def _zero_rows(ref, nrows, ncols):
    zv = jnp.zeros((16,), jnp.float32)

    def body(i, _):
        for j in range(ncols // 16):
            ref[i, pl.ds(j * 16, 16)] = zv
        return 0

    lax.fori_loop(0, nrows, body, 0)


def _fill_ones(ref, nrows, ncols):
    ov = jnp.ones((16,), jnp.float32)

    def body(i, _):
        for j in range(ncols // 16):
            ref[i, pl.ds(j * 16, 16)] = ov
        return 0

    lax.fori_loop(0, nrows, body, 0)


def _zero_shared(zero_v, shared, sid):
    def zbody(t, _):
        pltpu.sync_copy(zero_v, shared.at[pl.ds(sid * 625 + t * ZSP, ZSP)])
        return 0

    lax.fori_loop(0, 625 // ZSP, zbody, 0)


def _writeout_shared(shared, out2d, sid):
    pltpu.sync_copy(
        shared.at[pl.ds(sid * WPT, WPT)], out2d.at[pl.ds(sid * WPT, WPT)]
    )

    @pl.when(sid == NS - 1)
    def _():
        pltpu.sync_copy(
            shared.at[pl.ds(NS * WPT, N - NS * WPT)],
            out2d.at[pl.ds(NS * WPT, N - NS * WPT)],
        )


def _make_hist():
    mesh = plsc.VectorSubcoreMesh(core_axis_name="c", subcore_axis_name="s")

    @functools.partial(
        pl.kernel,
        mesh=mesh,
        out_type=jax.ShapeDtypeStruct((2, N, 16), jnp.float32),
        scratch_types=[
            pltpu.VMEM((NSEG, SEG, GK), jnp.int32),
            pltpu.VMEM((104, 16), jnp.float32),
            pltpu.VMEM((32, 16), jnp.float32),
            pltpu.VMEM_SHARED((N, 16), jnp.float32),
        ],
    )
    def hist(ei_hbm, oz_hbm, deg_hbm, idx_v, ones_v, zero_v, hist_s):
        cid = lax.axis_index("c")
        sid = lax.axis_index("s")
        pltpu.sync_copy(ei_hbm.at[cid, sid], idx_v)
        pltpu.sync_copy(oz_hbm.at[pl.ds(0, 104)], ones_v)
        pltpu.sync_copy(oz_hbm.at[pl.ds(104, 32)], zero_v)
        _zero_shared(zero_v.at[pl.ds(0, ZSP)], hist_s, sid)
        plsc.subcore_barrier()

        def body(g, _):
            pltpu.sync_copy(
                ones_v.at[pl.ds(0, GK)],
                hist_s.at[idx_v.at[g // SEG, g % SEG]],
                add=True,
            )
            return 0

        lax.fori_loop(0, G, body, 0)
        plsc.subcore_barrier()
        _writeout_shared(hist_s, deg_hbm.at[cid], sid)

    return hist


def _make_spmm(C):
    CPC = C // NC
    mesh = plsc.VectorSubcoreMesh(core_axis_name="c", subcore_axis_name="s")

    @functools.partial(
        pl.kernel,
        mesh=mesh,
        out_type=jax.ShapeDtypeStruct((C, N, FC), jnp.float32),
        scratch_types=[
            pltpu.VMEM((SEG, GK), jnp.int32),
            pltpu.VMEM((SEG, GK), jnp.int32),
            pltpu.VMEM((2, GK, FC), jnp.float32),
            pltpu.VMEM((ZSP, FC), jnp.float32),
            pltpu.VMEM_SHARED((N, FC), jnp.float32),
            pltpu.SemaphoreType.DMA,
            pltpu.SemaphoreType.DMA,
        ],
    )
    def spmm(h_hbm, ei_hbm, out_hbm, sidx, didx, buf, zero_v, agg, sem0, sem1):
        cid = lax.axis_index("c")
        sid = lax.axis_index("s")
        _zero_rows(zero_v, ZSP, FC)

        for lc in range(CPC):
            c = cid * CPC + lc
            _zero_shared(zero_v, agg, sid)
            plsc.subcore_barrier()

            def gat(g, slot, sem):
                return pltpu.async_copy(
                    h_hbm.at[c].at[sidx.at[g]], buf.at[slot], sem
                )

            def wait(g, slot, sem):
                pltpu.make_async_copy(
                    h_hbm.at[c].at[sidx.at[g]], buf.at[slot], sem
                ).wait()

            def sca(g, slot):
                pltpu.sync_copy(buf.at[slot], agg.at[didx.at[g]], add=True)

            def segbody(seg, _):
                pltpu.sync_copy(ei_hbm.at[0, sid, seg], sidx)
                pltpu.sync_copy(ei_hbm.at[1, sid, seg], didx)
                gat(0, 0, sem0)

                def body(gg, _):
                    g0 = 2 * gg
                    gat(g0 + 1, 1, sem1)
                    wait(g0, 0, sem0)
                    sca(g0, 0)
                    gat(g0 + 2, 0, sem0)
                    wait(g0 + 1, 1, sem1)
                    sca(g0 + 1, 1)
                    return 0

                lax.fori_loop(0, SEG // 2 - 1, body, 0)
                gat(SEG - 1, 1, sem1)
                wait(SEG - 2, 0, sem0)
                sca(SEG - 2, 0)
                wait(SEG - 1, 1, sem1)
                sca(SEG - 1, 1)
                return 0

            lax.fori_loop(0, NSEG, segbody, 0)

            plsc.subcore_barrier()
            _writeout_shared(agg, out_hbm.at[c], sid)
            if lc + 1 < CPC:
                plsc.subcore_barrier()

    return spmm


RB = 1000


def _prep_body(deg_ref, x_ref, ns_ref, nd_ref, p0_ref):
    ns = lax.rsqrt(jnp.maximum(deg_ref[0, :, 0:1], 1.0))
    nd = lax.rsqrt(jnp.maximum(deg_ref[1, :, 0:1], 1.0))
    ns_ref[...] = ns
    nd_ref[...] = nd
    x = x_ref[...] * ns
    p0_ref[0] = x[:, :FC]
    p0_ref[1] = x[:, FC:]


def _prep(deg, features):
    grid = (N // RB,)
    return pl.pallas_call(
        _prep_body,
        grid=grid,
        in_specs=[
            pl.BlockSpec((2, RB, 16), lambda i: (0, i, 0)),
            pl.BlockSpec((RB, 256), lambda i: (i, 0)),
        ],
        out_specs=[
            pl.BlockSpec((RB, 1), lambda i: (i, 0)),
            pl.BlockSpec((RB, 1), lambda i: (i, 0)),
            pl.BlockSpec((2, RB, FC), lambda i: (0, i, 0)),
        ],
        out_shape=[
            jax.ShapeDtypeStruct((N, 1), jnp.float32),
            jax.ShapeDtypeStruct((N, 1), jnp.float32),
            jax.ShapeDtypeStruct((2, N, FC), jnp.float32),
        ],
    )(deg, features)


def _mm_body(nd_ref, x_ref, w_ref, b_ref, ns_ref, o_ref, *, cin, cout, relu,
             scaled, bias):
    x = jnp.concatenate([x_ref[i] for i in range(cin)], axis=1)
    if scaled:
        x = x * nd_ref[...]
    y = jnp.dot(x, w_ref[...], preferred_element_type=jnp.float32)
    if bias:
        y = y + b_ref[...]
    if relu:
        y = jnp.maximum(y, 0.0)
    if scaled:
        y = y * ns_ref[...]
    for j in range(cout):
        o_ref[j] = y[:, j * FC:(j + 1) * FC]


def _mm(x, w, b, nd, ns, relu, scaled, bias):
    cin = x.shape[0]
    k, h = w.shape
    cout = h // FC
    grid = (N // RB,)
    body = functools.partial(_mm_body, cin=cin, cout=cout, relu=relu,
                             scaled=scaled, bias=bias)
    return pl.pallas_call(
        body,
        grid=grid,
        in_specs=[
            pl.BlockSpec((RB, 1), lambda i: (i, 0)),
            pl.BlockSpec((cin, RB, FC), lambda i: (0, i, 0)),
            pl.BlockSpec((k, h), lambda i: (0, 0)),
            pl.BlockSpec((1, h), lambda i: (0, 0)),
            pl.BlockSpec((RB, 1), lambda i: (i, 0)),
        ],
        out_specs=pl.BlockSpec((cout, RB, FC), lambda i: (0, i, 0)),
        out_shape=jax.ShapeDtypeStruct((cout, N, FC), jnp.float32),
    )(nd, x, w, b.reshape(1, h), ns)


def _mm23_body(nd_ref, x_ref, w2_ref, b2_ref, ns_ref, w3_ref, o_ref, *, cin,
               cout):
    x = jnp.concatenate([x_ref[i] for i in range(cin)], axis=1)
    x = x * nd_ref[...]
    z = jnp.dot(x, w2_ref[...], preferred_element_type=jnp.float32)
    z = jnp.maximum(z + b2_ref[...], 0.0) * ns_ref[...]
    y = jnp.dot(z, w3_ref[...], preferred_element_type=jnp.float32)
    for j in range(cout):
        o_ref[j] = y[:, j * FC:(j + 1) * FC]


def _mm23(x, w2, b2, w3, nd, ns):
    cin = x.shape[0]
    k, h = w2.shape
    h3 = w3.shape[1]
    cout = h3 // FC
    grid = (N // RB,)
    body = functools.partial(_mm23_body, cin=cin, cout=cout)
    return pl.pallas_call(
        body,
        grid=grid,
        in_specs=[
            pl.BlockSpec((RB, 1), lambda i: (i, 0)),
            pl.BlockSpec((cin, RB, FC), lambda i: (0, i, 0)),
            pl.BlockSpec((k, h), lambda i: (0, 0)),
            pl.BlockSpec((1, h), lambda i: (0, 0)),
            pl.BlockSpec((RB, 1), lambda i: (i, 0)),
            pl.BlockSpec((h, h3), lambda i: (0, 0)),
        ],
        out_specs=pl.BlockSpec((cout, RB, FC), lambda i: (0, i, 0)),
        out_shape=jax.ShapeDtypeStruct((cout, N, FC), jnp.float32),
    )(nd, x, w2, b2.reshape(1, h), ns, w3)


def _final_body(a_ref, nd_ref, b_ref, o_ref):
    o_ref[...] = jnp.maximum(a_ref[0] * nd_ref[...] + b_ref[...], 0.0)


def _final(a3, nd, b3):
    grid = (N // RB, 2)
    return pl.pallas_call(
        _final_body,
        grid=grid,
        in_specs=[
            pl.BlockSpec((1, RB, FC), lambda i, j: (j, i, 0)),
            pl.BlockSpec((RB, 1), lambda i, j: (i, 0)),
            pl.BlockSpec((1, FC), lambda i, j: (0, j)),
        ],
        out_specs=pl.BlockSpec((RB, FC), lambda i, j: (i, j)),
        out_shape=jax.ShapeDtypeStruct((N, 256), jnp.float32),
    )(a3, nd, b3.reshape(1, 256))


_hist_k = _make_hist()
_spmm2 = _make_spmm(2)
_spmm4 = _make_spmm(4)


@jax.jit
def kernel(features, edge_index, W1, b1, W2, b2, W3, b3):
    ei = edge_index.reshape(2, NS, NSEG, SEG, GK)
    oz = jnp.concatenate(
        [jnp.ones((GK, 16), jnp.float32), jnp.zeros((136 - GK, 16), jnp.float32)]
    )
    deg = _hist_k(ei, oz)
    ns, nd, p0 = _prep(deg, features)

    a1 = _spmm2(p0, ei)
    h1 = _mm(a1, W1, b1, nd, ns, relu=True, scaled=True, bias=True)
    a2 = _spmm4(h1, ei)
    p2 = _mm23(a2, W2, b2, W3, nd, ns)
    a3 = _spmm2(p2, ei)
    return _final(a3, nd, b3)


# speedup vs baseline: 9.7846x; 1.0484x over previous
import functools

import jax
import jax.numpy as jnp
from jax import lax
from jax.experimental import pallas as pl
from jax.experimental.pallas import tpu as pltpu
from jax.experimental.pallas import tpu_sc as plsc

N = 10000
E = 160000
FC = 128
NC = 2
NS = 16
EPT = E // NS
GK = 125
G = EPT // GK
SEG = 40
NSEG = G // SEG
ZSP = 25
WPT = 624


def _zero_rows(ref, nrows, ncols):
    zv = jnp.zeros((16,), jnp.float32)

    def body(i, _):
        for j in range(ncols // 16):
            ref[i, pl.ds(j * 16, 16)] = zv
        return 0

    lax.fori_loop(0, nrows, body, 0)


def _fill_ones(ref, nrows, ncols):
    ov = jnp.ones((16,), jnp.float32)

    def body(i, _):
        for j in range(ncols // 16):
            ref[i, pl.ds(j * 16, 16)] = ov
        return 0

    lax.fori_loop(0, nrows, body, 0)


def _zero_shared(zero_v, shared, sid):
    def zbody(t, _):
        pltpu.sync_copy(zero_v, shared.at[pl.ds(sid * 625 + t * ZSP, ZSP)])
        return 0

    lax.fori_loop(0, 625 // ZSP, zbody, 0)


def _zero_shared16(zero_v, shared, sid):
    def zbody(t, _):
        pltpu.sync_copy(zero_v, shared.at[pl.ds(sid * WPT + t * 48, 48)])
        return 0

    lax.fori_loop(0, WPT // 48, zbody, 0)

    @pl.when(sid == NS - 1)
    def _():
        pltpu.sync_copy(
            zero_v.at[pl.ds(0, N - NS * WPT)],
            shared.at[pl.ds(NS * WPT, N - NS * WPT)],
        )


def _writeout_shared(shared, out2d, sid):
    pltpu.sync_copy(
        shared.at[pl.ds(sid * WPT, WPT)], out2d.at[pl.ds(sid * WPT, WPT)]
    )

    @pl.when(sid == NS - 1)
    def _():
        pltpu.sync_copy(
            shared.at[pl.ds(NS * WPT, N - NS * WPT)],
            out2d.at[pl.ds(NS * WPT, N - NS * WPT)],
        )


def _make_hist():
    mesh = plsc.VectorSubcoreMesh(core_axis_name="c", subcore_axis_name="s")

    @functools.partial(
        pl.kernel,
        mesh=mesh,
        out_type=jax.ShapeDtypeStruct((2, N, 16), jnp.float32),
        scratch_types=[
            pltpu.VMEM((NSEG, SEG, GK), jnp.int32),
            pltpu.VMEM((104, 16), jnp.float32),
            pltpu.VMEM((32, 16), jnp.float32),
            pltpu.VMEM_SHARED((N, 16), jnp.float32),
        ],
    )
    def hist(ei_hbm, oz_hbm, deg_hbm, idx_v, ones_v, zero_v, hist_s):
        cid = lax.axis_index("c")
        sid = lax.axis_index("s")
        pltpu.sync_copy(ei_hbm.at[cid, sid], idx_v)
        pltpu.sync_copy(oz_hbm.at[pl.ds(0, 104)], ones_v)
        pltpu.sync_copy(oz_hbm.at[pl.ds(104, 32)], zero_v)
        _zero_shared(zero_v.at[pl.ds(0, ZSP)], hist_s, sid)
        plsc.subcore_barrier()

        def body(g, _):
            pltpu.sync_copy(
                ones_v.at[pl.ds(0, GK)],
                hist_s.at[idx_v.at[g // SEG, g % SEG]],
                add=True,
            )
            return 0

        lax.fori_loop(0, G, body, 0)
        plsc.subcore_barrier()
        _writeout_shared(hist_s, deg_hbm.at[cid], sid)

    return hist


def _make_spmm(C):
    CPC = C // NC
    mesh = plsc.VectorSubcoreMesh(core_axis_name="c", subcore_axis_name="s")

    @functools.partial(
        pl.kernel,
        mesh=mesh,
        out_type=jax.ShapeDtypeStruct((C, N, FC), jnp.float32),
        scratch_types=[
            pltpu.VMEM((SEG, GK), jnp.int32),
            pltpu.VMEM((SEG, GK), jnp.int32),
            pltpu.VMEM((2, GK, FC), jnp.float32),
            pltpu.VMEM((ZSP, FC), jnp.float32),
            pltpu.VMEM_SHARED((N, FC), jnp.float32),
            pltpu.SemaphoreType.DMA,
            pltpu.SemaphoreType.DMA,
        ],
    )
    def spmm(h_hbm, ei_hbm, out_hbm, sidx, didx, buf, zero_v, agg, sem0, sem1):
        cid = lax.axis_index("c")
        sid = lax.axis_index("s")
        _zero_rows(zero_v, ZSP, FC)

        for lc in range(CPC):
            c = cid * CPC + lc
            _zero_shared(zero_v, agg, sid)
            plsc.subcore_barrier()

            def gat(g, slot, sem):
                return pltpu.async_copy(
                    h_hbm.at[c].at[sidx.at[g]], buf.at[slot], sem
                )

            def wait(g, slot, sem):
                pltpu.make_async_copy(
                    h_hbm.at[c].at[sidx.at[g]], buf.at[slot], sem
                ).wait()

            def sca(g, slot):
                pltpu.sync_copy(buf.at[slot], agg.at[didx.at[g]], add=True)

            def segbody(seg, _):
                pltpu.sync_copy(ei_hbm.at[0, sid, seg], sidx)
                pltpu.sync_copy(ei_hbm.at[1, sid, seg], didx)
                gat(0, 0, sem0)

                def body(gg, _):
                    g0 = 2 * gg
                    gat(g0 + 1, 1, sem1)
                    wait(g0, 0, sem0)
                    sca(g0, 0)
                    gat(g0 + 2, 0, sem0)
                    wait(g0 + 1, 1, sem1)
                    sca(g0 + 1, 1)
                    return 0

                lax.fori_loop(0, SEG // 2 - 1, body, 0)
                gat(SEG - 1, 1, sem1)
                wait(SEG - 2, 0, sem0)
                sca(SEG - 2, 0)
                wait(SEG - 1, 1, sem1)
                sca(SEG - 1, 1)
                return 0

            lax.fori_loop(0, NSEG, segbody, 0)

            plsc.subcore_barrier()
            _writeout_shared(agg, out_hbm.at[c], sid)
            if lc + 1 < CPC:
                plsc.subcore_barrier()

    return spmm


RB = 1000
NB = N // RB


def _prep_body(deg_ref, x_ref, ns_ref, nd_ref, p0_ref):
    ns = lax.rsqrt(jnp.maximum(deg_ref[0, :, 0:1], 1.0))
    nd = lax.rsqrt(jnp.maximum(deg_ref[1, :, 0:1], 1.0))
    ns_ref[...] = ns
    nd_ref[...] = nd
    x = x_ref[...] * ns
    p0_ref[0] = x[:, :FC]
    p0_ref[1] = x[:, FC:]


def _prep(deg, features):
    return pl.pallas_call(
        _prep_body,
        grid=(NB,),
        in_specs=[
            pl.BlockSpec((2, RB, 16), lambda i: (0, i, 0)),
            pl.BlockSpec((RB, 256), lambda i: (i, 0)),
        ],
        out_specs=[
            pl.BlockSpec((RB, 1), lambda i: (i, 0)),
            pl.BlockSpec((RB, 1), lambda i: (i, 0)),
            pl.BlockSpec((2, RB, FC), lambda i: (0, i, 0)),
        ],
        out_shape=[
            jax.ShapeDtypeStruct((N, 1), jnp.float32),
            jax.ShapeDtypeStruct((N, 1), jnp.float32),
            jax.ShapeDtypeStruct((2, N, FC), jnp.float32),
        ],
    )(deg, features)


def _mm_body(nd_ref, x_ref, w_ref, b_ref, ns_ref, o_ref, *, cin, cout):
    x = jnp.concatenate([x_ref[i] for i in range(cin)], axis=1)
    x = x * nd_ref[...]
    y = jnp.dot(x, w_ref[...], preferred_element_type=jnp.float32)
    y = jnp.maximum(y + b_ref[...], 0.0) * ns_ref[...]
    for j in range(cout):
        o_ref[j] = y[:, j * FC:(j + 1) * FC]


def _mm(x, w, b, nd, ns):
    cin = x.shape[0]
    k, h = w.shape
    cout = h // FC
    body = functools.partial(_mm_body, cin=cin, cout=cout)
    return pl.pallas_call(
        body,
        grid=(NB,),
        in_specs=[
            pl.BlockSpec((RB, 1), lambda i: (i, 0)),
            pl.BlockSpec((cin, RB, FC), lambda i: (0, i, 0)),
            pl.BlockSpec((k, h), lambda i: (0, 0)),
            pl.BlockSpec((1, h), lambda i: (0, 0)),
            pl.BlockSpec((RB, 1), lambda i: (i, 0)),
        ],
        out_specs=pl.BlockSpec((cout, RB, FC), lambda i: (0, i, 0)),
        out_shape=jax.ShapeDtypeStruct((cout, N, FC), jnp.float32),
    )(nd, x, w, b.reshape(1, h), ns)


def _mm23_body(nd_ref, x_ref, w2_ref, b2_ref, ns_ref, w3_ref, o_ref, *, cin,
               cout):
    x = jnp.concatenate([x_ref[i] for i in range(cin)], axis=1)
    x = x * nd_ref[...]
    z = jnp.dot(x, w2_ref[...], preferred_element_type=jnp.float32)
    z = jnp.maximum(z + b2_ref[...], 0.0) * ns_ref[...]
    y = jnp.dot(z, w3_ref[...], preferred_element_type=jnp.float32)
    for j in range(cout):
        o_ref[j] = y[:, j * FC:(j + 1) * FC]


def _mm23(x, w2, b2, w3, nd, ns):
    cin = x.shape[0]
    k, h = w2.shape
    h3 = w3.shape[1]
    cout = h3 // FC
    body = functools.partial(_mm23_body, cin=cin, cout=cout)
    return pl.pallas_call(
        body,
        grid=(NB,),
        in_specs=[
            pl.BlockSpec((RB, 1), lambda i: (i, 0)),
            pl.BlockSpec((cin, RB, FC), lambda i: (0, i, 0)),
            pl.BlockSpec((k, h), lambda i: (0, 0)),
            pl.BlockSpec((1, h), lambda i: (0, 0)),
            pl.BlockSpec((RB, 1), lambda i: (i, 0)),
            pl.BlockSpec((h, h3), lambda i: (0, 0)),
        ],
        out_specs=pl.BlockSpec((cout, RB, FC), lambda i: (0, i, 0)),
        out_shape=jax.ShapeDtypeStruct((cout, N, FC), jnp.float32),
    )(nd, x, w2, b2.reshape(1, h), ns, w3)


def _final_body(a_ref, nd_ref, b_ref, o_ref):
    x = jnp.concatenate([a_ref[0], a_ref[1]], axis=1)
    o_ref[...] = jnp.maximum(x * nd_ref[...] + b_ref[...], 0.0)


def _final(a3, nd, b3):
    return pl.pallas_call(
        _final_body,
        grid=(NB,),
        in_specs=[
            pl.BlockSpec((2, RB, FC), lambda i: (0, i, 0)),
            pl.BlockSpec((RB, 1), lambda i: (i, 0)),
            pl.BlockSpec((1, 256), lambda i: (0, 0)),
        ],
        out_specs=pl.BlockSpec((RB, 256), lambda i: (i, 0)),
        out_shape=jax.ShapeDtypeStruct((N, 256), jnp.float32),
    )(a3, nd, b3.reshape(1, 256))


_hist_k = _make_hist()
_spmm2 = _make_spmm(2)
_spmm4 = _make_spmm(4)


@jax.jit
def kernel(features, edge_index, W1, b1, W2, b2, W3, b3):
    ei = edge_index.reshape(2, NS, NSEG, SEG, GK)
    oz = jnp.concatenate(
        [jnp.ones((GK, 16), jnp.float32), jnp.zeros((136 - GK, 16), jnp.float32)]
    )
    deg = _hist_k(ei, oz)
    ns, nd, p0 = _prep(deg, features)

    a1 = _spmm2(p0, ei)
    h1 = _mm(a1, W1, b1, nd, ns)
    a2 = _spmm4(h1, ei)
    p2 = _mm23(a2, W2, b2, W3, nd, ns)
    a3 = _spmm2(p2, ei)
    return _final(a3, nd, b3)
